# baseline (speedup 1.0000x reference)
import jax
import jax.numpy as jnp
from jax import lax
from jax.experimental import pallas as pl
from jax.experimental.pallas import tpu as pltpu
from jax.experimental.pallas import tpu_sc as plsc

_BIG = 100000.0
_T = 100
_TP = 128
_DIM = 32
_ROWS = 10000
_BR = 2000
_NW = 8
_BPW = _TP // _NW
_GPAD = 100096
_INV_SQRT2 = 0.7071067811865476


def _sc_gather_body(zc4_hbm, gpad_hbm, izc_hbm, cb_hbm, igc_hbm, mg_hbm,
                    zc_out, gc_out,
                    izv, cbv, igv, mgv, zrows_v, grows_v, zc_v, gcol_v, sem):
    wid = lax.axis_index("s") * 2 + lax.axis_index("c")

    @pl.when(wid < _NW)
    def _():
        base = wid * _BPW
        pltpu.sync_copy(izc_hbm.at[pl.ds(base, _BPW)], izv)
        pltpu.sync_copy(cb_hbm.at[pl.ds(base, _BPW)], cbv)
        pltpu.sync_copy(igc_hbm.at[pl.ds(base, _BPW)], igv)
        pltpu.sync_copy(mg_hbm.at[pl.ds(base, _BPW)], mgv)
        pltpu.async_copy(zc4_hbm.at[izv], zrows_v, sem).wait()
        pltpu.async_copy(gpad_hbm.at[igv], grows_v, sem).wait()
        rowi = lax.iota(jnp.int32, 16)
        cb = cbv[...]
        for d in range(_DIM):
            val = plsc.load_gather(zrows_v, [rowi, cb + d])
            plsc.store_scatter(zc_v, [rowi, jnp.full((16,), d, jnp.int32)],
                               val)
        gcol_v[...] = plsc.load_gather(grows_v, [rowi, mgv[...]])
        pltpu.sync_copy(zc_v, zc_out.at[pl.ds(base, _BPW)])
        pltpu.sync_copy(gcol_v, gc_out.at[pl.ds(base, _BPW)])


def _sc_gather(zc4, gpad2, izc, cb, igc, mg):
    return pl.kernel(
        _sc_gather_body,
        out_type=(
            jax.ShapeDtypeStruct((_TP, _DIM), jnp.float32),
            jax.ShapeDtypeStruct((_TP,), jnp.float32),
        ),
        mesh=plsc.VectorSubcoreMesh(core_axis_name="c", subcore_axis_name="s"),
        compiler_params=pltpu.CompilerParams(needs_layout_passes=False),
        scratch_types=[
            pltpu.VMEM((_BPW,), jnp.int32),
            pltpu.VMEM((_BPW,), jnp.int32),
            pltpu.VMEM((_BPW,), jnp.int32),
            pltpu.VMEM((_BPW,), jnp.int32),
            pltpu.VMEM((_BPW, 128), jnp.float32),
            pltpu.VMEM((_BPW, 128), jnp.float32),
            pltpu.VMEM((_BPW, _DIM), jnp.float32),
            pltpu.VMEM((_BPW,), jnp.float32),
            pltpu.SemaphoreType.DMA,
        ],
    )(zc4, gpad2, izc, cb, igc, mg)


def _tc_body(mat_ref, tcol_ref, gr_ref, gc_ref, zc_ref, zr_ref,
             theta_ref, sigma_ref, out_ref):
    i = pl.program_id(0)

    @pl.when(i == 0)
    def _():
        out_ref[0, 0] = 0.0

    tc1 = tcol_ref[...]
    tc2 = 0.5 * tc1 * tc1
    wv = zc_ref[...] - 1e-6
    ww = jnp.sum(wv * wv, axis=1, keepdims=True)
    p00 = jnp.ones_like(tc1)
    Wt = jnp.concatenate([
        -2.0 * wv, (-2.0 * tc1) * wv, (-2.0 * tc2) * wv,
        p00, 2.0 * tc1, 2.0 * tc2, tc1 * tc1, 2.0 * tc1 * tc2, tc2 * tc2,
        ww,
    ], axis=1)

    z0 = zr_ref[0]
    z1 = zr_ref[1]
    z2 = zr_ref[2]
    q00 = jnp.sum(z0 * z0, axis=1, keepdims=True)
    q01 = jnp.sum(z0 * z1, axis=1, keepdims=True)
    q02 = jnp.sum(z0 * z2, axis=1, keepdims=True)
    q11 = jnp.sum(z1 * z1, axis=1, keepdims=True)
    q12 = jnp.sum(z1 * z2, axis=1, keepdims=True)
    q22 = jnp.sum(z2 * z2, axis=1, keepdims=True)
    F = jnp.concatenate([
        z0, z1, z2, q00, q01, q02, q11, q12, q22, jnp.ones_like(q00),
    ], axis=1)

    dist2 = lax.dot_general(F, Wt, (((1,), (1,)), ((), ())),
                            preferred_element_type=jnp.float32,
                            precision=lax.Precision.HIGHEST)
    dist = jnp.sqrt(jnp.maximum(dist2[:, :_T], 0.0))

    f = -dist + gr_ref[...] + gc_ref[...]

    mat = mat_ref[...]
    active = mat != 0
    y1 = jnp.where(active, mat, 1)
    th = [theta_ref[k] for k in range(6)]
    thi = jnp.where(y1 == 1, th[1],
          jnp.where(y1 == 2, th[2],
          jnp.where(y1 == 3, th[3], th[4])))
    tlo = jnp.where(y1 == 1, th[0],
          jnp.where(y1 == 2, th[1],
          jnp.where(y1 == 3, th[2], th[3])))

    inv_sigma = 1.0 / sigma_ref[0]
    cdf_hi = 0.5 * (1.0 + lax.erf((thi - f) * inv_sigma * _INV_SQRT2))
    cdf_lo = 0.5 * (1.0 + lax.erf((tlo - f) * inv_sigma * _INV_SQRT2))
    ll = jnp.log(cdf_hi - cdf_lo)
    ll = jnp.where(active, ll, 0.0)
    out_ref[0, 0] += jnp.sum(ll)


def _tc_call(mat, tcol, gr2d, gc2d, zc, z_rows, theta, sigma):
    grid = (_ROWS // _BR,)
    return pl.pallas_call(
        _tc_body,
        grid=grid,
        in_specs=[
            pl.BlockSpec((_BR, _T), lambda i: (i, 0)),
            pl.BlockSpec((_TP, 1), lambda i: (0, 0)),
            pl.BlockSpec((_BR, 1), lambda i: (i, 0)),
            pl.BlockSpec((1, _T), lambda i: (0, 0)),
            pl.BlockSpec((_TP, _DIM), lambda i: (0, 0)),
            pl.BlockSpec((3, _BR, _DIM), lambda i: (0, i, 0)),
            pl.BlockSpec(memory_space=pltpu.SMEM),
            pl.BlockSpec(memory_space=pltpu.SMEM),
        ],
        out_specs=pl.BlockSpec((1, 1), lambda i: (0, 0),
                               memory_space=pltpu.SMEM),
        out_shape=jax.ShapeDtypeStruct((1, 1), jnp.float32),
    )(mat, tcol, gr2d, gc2d, zc, z_rows, theta, sigma)


def kernel(batch_events_mat, col_idx_list, batch_events_time,
           gamma_rows, gamma_cols, z_rows, z_cols, b, sigma):
    idx = jnp.pad(col_idx_list.astype(jnp.int32), (0, _TP - _T))
    zc4 = z_cols.reshape(25000, 128)
    gpad2 = jnp.pad(gamma_cols, (0, _GPAD - gamma_cols.shape[0])
                    ).reshape(_GPAD // 128, 128)
    izc = idx // 4
    cb = (idx % 4) * _DIM
    igc = idx // 128
    mg = idx % 128
    zc, gc = _sc_gather(zc4, gpad2, izc, cb, igc, mg)
    gc2d = gc[:_T].reshape(1, _T)
    tcol = jnp.pad(batch_events_time, (0, _TP - _T)).reshape(_TP, 1)
    gr2d = gamma_rows.reshape(-1, 1)
    theta = jnp.concatenate([
        jnp.array([-_BIG], jnp.float32),
        b.astype(jnp.float32),
        jnp.array([_BIG], jnp.float32),
    ])
    total = _tc_call(batch_events_mat.astype(jnp.int32), tcol, gr2d,
                     gc2d, zc, z_rows, theta, sigma)
    return -total[0, 0]


# speedup vs baseline: 1.1826x; 1.1826x over previous
import jax
import jax.numpy as jnp
from jax import lax
from jax.experimental import pallas as pl
from jax.experimental.pallas import tpu as pltpu
from jax.experimental.pallas import tpu_sc as plsc

_BIG = 100000.0
_T = 100
_TP = 128
_DIM = 32
_ROWS = 10000
_BR = 2048
_NW = 8
_BPW = _TP // _NW
_GPAD = 100096
_INV_SQRT2 = 0.7071067811865476


def _sc_gather_body(gpad_hbm, idx_hbm, gc_out, idxv, igv, grows_v, gcol_v,
                    sem):
    wid = lax.axis_index("s") * 2 + lax.axis_index("c")

    @pl.when(wid < _NW)
    def _():
        base = wid * _BPW
        pltpu.sync_copy(idx_hbm.at[pl.ds(base, _BPW)], idxv)
        iv = idxv[...]
        igv[...] = lax.shift_right_logical(iv, 7)
        pltpu.async_copy(gpad_hbm.at[igv], grows_v, sem).wait()
        rowi = lax.iota(jnp.int32, 16)
        gcol_v[...] = plsc.load_gather(grows_v, [rowi, iv & 127])
        pltpu.sync_copy(gcol_v, gc_out.at[pl.ds(base, _BPW)])


def _sc_gather(gpad2, idx):
    return pl.kernel(
        _sc_gather_body,
        out_type=jax.ShapeDtypeStruct((_TP,), jnp.float32),
        mesh=plsc.VectorSubcoreMesh(core_axis_name="c", subcore_axis_name="s"),
        compiler_params=pltpu.CompilerParams(needs_layout_passes=False),
        scratch_types=[
            pltpu.VMEM((_BPW,), jnp.int32),
            pltpu.VMEM((_BPW,), jnp.int32),
            pltpu.VMEM((_BPW, 128), jnp.float32),
            pltpu.VMEM((_BPW,), jnp.float32),
            pltpu.SemaphoreType.DMA,
        ],
    )(gpad2, idx)


def _tc_body(mat_ref, tcol_ref, gc_ref, zr_ref, gr_ref, zcols_hbm, idx_ref,
             b_ref, sigma_ref, out_ref, zc_vmem, zsem):
    i = pl.program_id(0)

    @pl.when(i == 0)
    def _():
        out_ref[0, 0] = 0.0
        copies = [
            pltpu.make_async_copy(
                zcols_hbm.at[pl.ds(idx_ref[j], 1)],
                zc_vmem.at[pl.ds(j, 1)], zsem)
            for j in range(_TP)
        ]
        for c in copies:
            c.start()
        for c in copies:
            c.wait()

    grcol = jnp.transpose(gr_ref[...])

    tc1 = tcol_ref[...]
    tc2 = 0.5 * tc1 * tc1
    wv = zc_vmem[...] - 1e-6
    ww = jnp.sum(wv * wv, axis=1, keepdims=True)
    ones_blk = jnp.ones((_TP, _DIM), jnp.float32)
    Wt = jnp.concatenate([
        -2.0 * wv, (-2.0 * tc1) * wv, (-2.0 * tc2) * wv,
        ones_blk, (2.0 * tc1) * ones_blk, (2.0 * tc2) * ones_blk,
        (tc1 * tc1) * ones_blk, (2.0 * tc1 * tc2) * ones_blk,
        (tc2 * tc2) * ones_blk,
        ww,
    ], axis=1)

    z0 = zr_ref[0]
    z1 = zr_ref[1]
    z2 = zr_ref[2]
    F = jnp.concatenate([
        z0, z1, z2, z0 * z0, z0 * z1, z0 * z2, z1 * z1, z1 * z2, z2 * z2,
        jnp.ones((_BR, 1), jnp.float32),
    ], axis=1)

    dist2 = lax.dot_general(F, Wt, (((1,), (1,)), ((), ())),
                            preferred_element_type=jnp.float32,
                            precision=lax.Precision.HIGHEST)
    dist = jnp.sqrt(jnp.maximum(dist2[:, :_T], 0.0))

    f = -dist + grcol + gc_ref[...]

    mat = mat_ref[...]
    rowid = lax.broadcasted_iota(jnp.int32, (_BR, _T), 0)
    active = (mat != 0) & (rowid < _ROWS - i * _BR)
    y1 = jnp.where(active, mat, 1)
    th = [-_BIG, b_ref[0], b_ref[1], b_ref[2], b_ref[3], _BIG]
    thi = jnp.where(y1 == 1, th[1],
          jnp.where(y1 == 2, th[2],
          jnp.where(y1 == 3, th[3], th[4])))
    tlo = jnp.where(y1 == 1, th[0],
          jnp.where(y1 == 2, th[1],
          jnp.where(y1 == 3, th[2], th[3])))

    inv_sigma = 1.0 / sigma_ref[0]
    cdf_hi = 0.5 * (1.0 + lax.erf((thi - f) * inv_sigma * _INV_SQRT2))
    cdf_lo = 0.5 * (1.0 + lax.erf((tlo - f) * inv_sigma * _INV_SQRT2))
    ll = jnp.log(cdf_hi - cdf_lo)
    ll = jnp.where(active, ll, 0.0)
    out_ref[0, 0] += jnp.sum(ll)


def _tc_call(mat, tcol, gc2d, z_rows, gamma_rows, z_cols, idx, b, sigma):
    grid = (pl.cdiv(_ROWS, _BR),)
    return pl.pallas_call(
        _tc_body,
        grid=grid,
        in_specs=[
            pl.BlockSpec((_BR, _T), lambda i: (i, 0)),
            pl.BlockSpec((_TP, 1), lambda i: (0, 0)),
            pl.BlockSpec((1, _T), lambda i: (0, 0)),
            pl.BlockSpec((3, _BR, _DIM), lambda i: (0, i, 0)),
            pl.BlockSpec((1, _BR), lambda i: (0, i)),
            pl.BlockSpec(memory_space=pltpu.MemorySpace.HBM),
            pl.BlockSpec(memory_space=pltpu.SMEM),
            pl.BlockSpec(memory_space=pltpu.SMEM),
            pl.BlockSpec(memory_space=pltpu.SMEM),
        ],
        out_specs=pl.BlockSpec((1, 1), lambda i: (0, 0),
                               memory_space=pltpu.SMEM),
        out_shape=jax.ShapeDtypeStruct((1, 1), jnp.float32),
        scratch_shapes=[
            pltpu.VMEM((_TP, _DIM), jnp.float32),
            pltpu.SemaphoreType.DMA,
        ],
    )(mat, tcol, gc2d, z_rows, gamma_rows, z_cols, idx, b, sigma)


def kernel(batch_events_mat, col_idx_list, batch_events_time,
           gamma_rows, gamma_cols, z_rows, z_cols, b, sigma):
    idx = jnp.pad(col_idx_list.astype(jnp.int32), (0, _TP - _T))
    gpad2 = jnp.pad(gamma_cols, (0, _GPAD - gamma_cols.shape[0])
                    ).reshape(_GPAD // 128, 128)
    gc = _sc_gather(gpad2, idx)
    gc2d = gc[:_T].reshape(1, _T)
    tcol = jnp.pad(batch_events_time, (0, _TP - _T)).reshape(_TP, 1)
    npad = pl.cdiv(_ROWS, _BR) * _BR - _ROWS
    gr_row = jnp.pad(gamma_rows, (0, npad)).reshape(1, -1)
    total = _tc_call(batch_events_mat.astype(jnp.int32), tcol, gc2d,
                     z_rows, gr_row, z_cols, idx,
                     b.astype(jnp.float32), sigma)
    return -total[0, 0]


# speedup vs baseline: 1.6549x; 1.3994x over previous
import jax
import jax.numpy as jnp
from jax import lax
from jax.experimental import pallas as pl
from jax.experimental.pallas import tpu as pltpu
from jax.experimental.pallas import tpu_sc as plsc

_BIG = 100000.0
_T = 100
_TP = 128
_DIM = 32
_ROWS = 10000
_BR = 2048
_NW = 8
_BPW = _TP // _NW
_GPAD = 100096
_COLS = 100000
_INV_SQRT2 = 0.7071067811865476


def _sc_gather_body(gpad_hbm, idx_hbm, gc_out, idxv, igv, grows_v, gcol_v,
                    sem):
    wid = lax.axis_index("s") * 2 + lax.axis_index("c")

    @pl.when(wid < _NW)
    def _():
        base = wid * _BPW
        pltpu.sync_copy(idx_hbm.at[pl.ds(base, _BPW)], idxv)
        iv = idxv[...]
        igv[...] = lax.shift_right_logical(iv, 7)
        pltpu.async_copy(gpad_hbm.at[igv], grows_v, sem).wait()
        rowi = lax.iota(jnp.int32, 16)
        gcol_v[...] = plsc.load_gather(grows_v, [rowi, iv & 127])
        pltpu.sync_copy(gcol_v, gc_out.at[pl.ds(base, _BPW)])


def _sc_gather(gpad2, idx):
    return pl.kernel(
        _sc_gather_body,
        out_type=jax.ShapeDtypeStruct((_TP,), jnp.float32),
        mesh=plsc.VectorSubcoreMesh(core_axis_name="c", subcore_axis_name="s"),
        compiler_params=pltpu.CompilerParams(needs_layout_passes=False),
        scratch_types=[
            pltpu.VMEM((_BPW,), jnp.int32),
            pltpu.VMEM((_BPW,), jnp.int32),
            pltpu.VMEM((_BPW, 128), jnp.float32),
            pltpu.VMEM((_BPW,), jnp.float32),
            pltpu.SemaphoreType.DMA,
        ],
    )(gpad2, idx)


def _tc_body(mat_ref, trow_ref, gc_ref, zr_ref, gr_ref, zcolsT_hbm, idx_ref,
             b_ref, sigma_ref, out_ref, zcT_vmem, gc_vmem, zsem):
    i = pl.program_id(0)

    @pl.when(i == 0)
    def _():
        out_ref[0, 0] = 0.0
        nlast = _COLS - 128
        copies = []
        for j in range(_TP):
            cb = jnp.minimum((idx_ref[j] >> 7) * 128, nlast)
            copies.append(pltpu.make_async_copy(
                zcolsT_hbm.at[:, pl.ds(pl.multiple_of(cb, 128), 128)],
                gc_vmem.at[j], zsem))
        for c in copies:
            c.start()
        for c in copies:
            c.wait()
        lane = lax.broadcasted_iota(jnp.int32, (_DIM, 128), 1)
        for j in range(_TP):
            cb = jnp.minimum((idx_ref[j] >> 7) * 128, nlast)
            m = idx_ref[j] - cb
            col = jnp.sum(jnp.where(lane == m, gc_vmem[j], 0.0),
                          axis=1, keepdims=True)
            zcT_vmem[:, pl.ds(j, 1)] = col

    grcol = jnp.transpose(gr_ref[...])

    tr1 = trow_ref[...]
    tr2 = 0.5 * tr1 * tr1
    wvT = zcT_vmem[...] - 1e-6
    ww = jnp.sum(wvT * wvT, axis=0, keepdims=True)
    ones_blk = jnp.ones((_DIM, _TP), jnp.float32)
    W = jnp.concatenate([
        -2.0 * wvT, (-2.0 * tr1) * wvT, (-2.0 * tr2) * wvT,
        ones_blk, (2.0 * tr1) * ones_blk, (2.0 * tr2) * ones_blk,
        (tr1 * tr1) * ones_blk, (2.0 * tr1 * tr2) * ones_blk,
        (tr2 * tr2) * ones_blk,
        ww,
    ], axis=0)

    z0 = zr_ref[0]
    z1 = zr_ref[1]
    z2 = zr_ref[2]
    F = jnp.concatenate([
        z0, z1, z2, z0 * z0, z0 * z1, z0 * z2, z1 * z1, z1 * z2, z2 * z2,
        jnp.ones((_BR, 1), jnp.float32),
    ], axis=1)

    dist2 = lax.dot_general(F, W, (((1,), (0,)), ((), ())),
                            preferred_element_type=jnp.float32,
                            precision=lax.Precision.HIGHEST)
    dist = jnp.sqrt(jnp.maximum(dist2[:, :_T], 0.0))

    f = -dist + grcol + gc_ref[...]

    mat = mat_ref[...]
    rowid = lax.broadcasted_iota(jnp.int32, (_BR, _T), 0)
    active = (mat != 0) & (rowid < _ROWS - i * _BR)
    y1 = jnp.where(active, mat, 1)
    th = [-_BIG, b_ref[0], b_ref[1], b_ref[2], b_ref[3], _BIG]
    thi = jnp.where(y1 == 1, th[1],
          jnp.where(y1 == 2, th[2],
          jnp.where(y1 == 3, th[3], th[4])))
    tlo = jnp.where(y1 == 1, th[0],
          jnp.where(y1 == 2, th[1],
          jnp.where(y1 == 3, th[2], th[3])))

    inv_sigma = 1.0 / sigma_ref[0]
    cdf_hi = 0.5 * (1.0 + lax.erf((thi - f) * inv_sigma * _INV_SQRT2))
    cdf_lo = 0.5 * (1.0 + lax.erf((tlo - f) * inv_sigma * _INV_SQRT2))
    ll = jnp.log(cdf_hi - cdf_lo)
    ll = jnp.where(active, ll, 0.0)
    out_ref[0, 0] += jnp.sum(ll)


def _tc_call(mat, trow, gc2d, z_rows, gamma_rows, z_colsT, idx, b, sigma):
    grid = (pl.cdiv(_ROWS, _BR),)
    return pl.pallas_call(
        _tc_body,
        grid=grid,
        in_specs=[
            pl.BlockSpec((_BR, _T), lambda i: (i, 0)),
            pl.BlockSpec((1, _TP), lambda i: (0, 0)),
            pl.BlockSpec((1, _T), lambda i: (0, 0)),
            pl.BlockSpec((3, _BR, _DIM), lambda i: (0, i, 0)),
            pl.BlockSpec((1, _BR), lambda i: (0, i)),
            pl.BlockSpec(memory_space=pltpu.MemorySpace.HBM),
            pl.BlockSpec(memory_space=pltpu.SMEM),
            pl.BlockSpec(memory_space=pltpu.SMEM),
            pl.BlockSpec(memory_space=pltpu.SMEM),
        ],
        out_specs=pl.BlockSpec((1, 1), lambda i: (0, 0),
                               memory_space=pltpu.SMEM),
        out_shape=jax.ShapeDtypeStruct((1, 1), jnp.float32),
        scratch_shapes=[
            pltpu.VMEM((_DIM, _TP), jnp.float32),
            pltpu.VMEM((_TP, _DIM, 128), jnp.float32),
            pltpu.SemaphoreType.DMA,
        ],
    )(mat, trow, gc2d, z_rows, gamma_rows, z_colsT, idx, b, sigma)


def kernel(batch_events_mat, col_idx_list, batch_events_time,
           gamma_rows, gamma_cols, z_rows, z_cols, b, sigma):
    idx = jnp.pad(col_idx_list.astype(jnp.int32), (0, _TP - _T))
    gpad2 = jnp.pad(gamma_cols, (0, _GPAD - gamma_cols.shape[0])
                    ).reshape(_GPAD // 128, 128)
    gc = _sc_gather(gpad2, idx)
    gc2d = gc[:_T].reshape(1, _T)
    trow = jnp.pad(batch_events_time, (0, _TP - _T)).reshape(1, _TP)
    npad = pl.cdiv(_ROWS, _BR) * _BR - _ROWS
    gr_row = jnp.pad(gamma_rows, (0, npad)).reshape(1, -1)
    total = _tc_call(batch_events_mat.astype(jnp.int32), trow, gc2d,
                     z_rows, gr_row, z_cols.T, idx,
                     b.astype(jnp.float32), sigma)
    return -total[0, 0]


# speedup vs baseline: 1.8805x; 1.1363x over previous
import jax
import jax.numpy as jnp
from jax import lax
from jax.experimental import pallas as pl
from jax.experimental.pallas import tpu as pltpu
from jax.experimental.pallas import tpu_sc as plsc

_BIG = 100000.0
_T = 100
_TP = 128
_DIM = 32
_ROWS = 10000
_BR = 2048
_NW = 8
_BPW = _TP // _NW
_GPAD = 100096
_COLS = 100000
_INV_SQRT2 = 0.7071067811865476


def _sc_gather_body(gpad_hbm, idx_hbm, gc_out, idxv, igv, grows_v, gcol_v,
                    sem):
    wid = lax.axis_index("s") * 2 + lax.axis_index("c")

    @pl.when(wid < _NW)
    def _():
        base = wid * _BPW
        pltpu.sync_copy(idx_hbm.at[pl.ds(base, _BPW)], idxv)
        iv = idxv[...]
        igv[...] = lax.shift_right_logical(iv, 7)
        pltpu.async_copy(gpad_hbm.at[igv], grows_v, sem).wait()
        rowi = lax.iota(jnp.int32, 16)
        gcol_v[...] = plsc.load_gather(grows_v, [rowi, iv & 127])
        pltpu.sync_copy(gcol_v, gc_out.at[pl.ds(base, _BPW)])


def _sc_gather(gpad2, idx):
    return pl.kernel(
        _sc_gather_body,
        out_type=jax.ShapeDtypeStruct((_TP,), jnp.float32),
        mesh=plsc.VectorSubcoreMesh(core_axis_name="c", subcore_axis_name="s"),
        compiler_params=pltpu.CompilerParams(needs_layout_passes=False),
        scratch_types=[
            pltpu.VMEM((_BPW,), jnp.int32),
            pltpu.VMEM((_BPW,), jnp.int32),
            pltpu.VMEM((_BPW, 128), jnp.float32),
            pltpu.VMEM((_BPW,), jnp.float32),
            pltpu.SemaphoreType.DMA,
        ],
    )(gpad2, idx)


def _tc_body(mat_hbm, trow_ref, gc_ref, zr_hbm, gr_ref, zcolsT_hbm, idx_ref,
             b_ref, sigma_ref, out_ref, zcT_vmem, gc_vmem, mat_buf, zr_buf,
             zsem, msems, zrsems):
    i = pl.program_id(0)
    ngrid = pl.num_programs(0)
    tail_rows = _ROWS - (_ROWS // _BR) * _BR

    def block_copies(k, b, rows):
        cs = [pltpu.make_async_copy(mat_hbm.at[pl.ds(k * _BR, rows)],
                                    mat_buf.at[b, pl.ds(0, rows)],
                                    msems.at[b])]
        for v in range(3):
            cs.append(pltpu.make_async_copy(zr_hbm.at[v, pl.ds(k * _BR, rows)],
                                            zr_buf.at[b, v, pl.ds(0, rows)],
                                            zrsems.at[b]))
        return cs

    @pl.when(i == 0)
    def _():
        for c in block_copies(0, 0, _BR):
            c.start()

    @pl.when(i < ngrid - 2)
    def _():
        for c in block_copies(i + 1, (i + 1) % 2, _BR):
            c.start()

    @pl.when(i == ngrid - 2)
    def _():
        for c in block_copies(ngrid - 1, (ngrid - 1) % 2, tail_rows):
            c.start()

    bsel = i % 2

    @pl.when(i < ngrid - 1)
    def _():
        for c in block_copies(i, bsel, _BR):
            c.wait()

    @pl.when(i == ngrid - 1)
    def _():
        for c in block_copies(i, bsel, tail_rows):
            c.wait()

    @pl.when(i == 0)
    def _():
        out_ref[0, 0] = 0.0
        nlast = _COLS - 128
        copies = []
        for j in range(_TP):
            cb = jnp.minimum((idx_ref[j] >> 7) * 128, nlast)
            copies.append(pltpu.make_async_copy(
                zcolsT_hbm.at[:, pl.ds(pl.multiple_of(cb, 128), 128)],
                gc_vmem.at[j], zsem))
        for c in copies:
            c.start()
        for c in copies:
            c.wait()
        lane = lax.broadcasted_iota(jnp.int32, (_DIM, 128), 1)
        for j in range(_TP):
            cb = jnp.minimum((idx_ref[j] >> 7) * 128, nlast)
            m = idx_ref[j] - cb
            col = jnp.sum(jnp.where(lane == m, gc_vmem[j], 0.0),
                          axis=1, keepdims=True)
            zcT_vmem[:, pl.ds(j, 1)] = col

    grcol = jnp.transpose(gr_ref[...])

    tr1 = trow_ref[...]
    tr2 = 0.5 * tr1 * tr1
    wvT = zcT_vmem[...] - 1e-6
    ww = jnp.sum(wvT * wvT, axis=0, keepdims=True)
    ones_blk = jnp.ones((_DIM, _TP), jnp.float32)
    W = jnp.concatenate([
        -2.0 * wvT, (-2.0 * tr1) * wvT, (-2.0 * tr2) * wvT,
        ones_blk, (2.0 * tr1) * ones_blk, (2.0 * tr2) * ones_blk,
        (tr1 * tr1) * ones_blk, (2.0 * tr1 * tr2) * ones_blk,
        (tr2 * tr2) * ones_blk,
        ww,
    ], axis=0)

    z0 = zr_buf[bsel, 0]
    z1 = zr_buf[bsel, 1]
    z2 = zr_buf[bsel, 2]
    F = jnp.concatenate([
        z0, z1, z2, z0 * z0, z0 * z1, z0 * z2, z1 * z1, z1 * z2, z2 * z2,
        jnp.ones((_BR, 1), jnp.float32),
    ], axis=1)

    dist2 = lax.dot_general(F, W, (((1,), (0,)), ((), ())),
                            preferred_element_type=jnp.float32,
                            precision=None)
    dist = jnp.sqrt(jnp.maximum(dist2[:, :_T], 0.0))

    f = -dist + grcol + gc_ref[...]

    mat = mat_buf[bsel]
    rowid = lax.broadcasted_iota(jnp.int32, (_BR, _T), 0)
    active = (mat != 0) & (rowid < _ROWS - i * _BR)
    y1 = jnp.where(active, mat, 1)
    th = [-_BIG, b_ref[0], b_ref[1], b_ref[2], b_ref[3], _BIG]
    thi = jnp.where(y1 == 1, th[1],
          jnp.where(y1 == 2, th[2],
          jnp.where(y1 == 3, th[3], th[4])))
    tlo = jnp.where(y1 == 1, th[0],
          jnp.where(y1 == 2, th[1],
          jnp.where(y1 == 3, th[2], th[3])))

    inv_sigma = 1.0 / sigma_ref[0]
    cdf_hi = 0.5 * (1.0 + lax.erf((thi - f) * inv_sigma * _INV_SQRT2))
    cdf_lo = 0.5 * (1.0 + lax.erf((tlo - f) * inv_sigma * _INV_SQRT2))
    ll = jnp.log(cdf_hi - cdf_lo)
    ll = jnp.where(active, ll, 0.0)
    out_ref[0, 0] += jnp.sum(ll)


def _tc_call(mat, trow, gc2d, z_rows, gamma_rows, z_colsT, idx, b, sigma):
    grid = (pl.cdiv(_ROWS, _BR),)
    return pl.pallas_call(
        _tc_body,
        grid=grid,
        in_specs=[
            pl.BlockSpec(memory_space=pltpu.MemorySpace.HBM),
            pl.BlockSpec((1, _TP), lambda i: (0, 0)),
            pl.BlockSpec((1, _T), lambda i: (0, 0)),
            pl.BlockSpec(memory_space=pltpu.MemorySpace.HBM),
            pl.BlockSpec((1, _BR), lambda i: (0, i)),
            pl.BlockSpec(memory_space=pltpu.MemorySpace.HBM),
            pl.BlockSpec(memory_space=pltpu.SMEM),
            pl.BlockSpec(memory_space=pltpu.SMEM),
            pl.BlockSpec(memory_space=pltpu.SMEM),
        ],
        out_specs=pl.BlockSpec((1, 1), lambda i: (0, 0),
                               memory_space=pltpu.SMEM),
        out_shape=jax.ShapeDtypeStruct((1, 1), jnp.float32),
        scratch_shapes=[
            pltpu.VMEM((_DIM, _TP), jnp.float32),
            pltpu.VMEM((_TP, _DIM, 128), jnp.float32),
            pltpu.VMEM((2, _BR, _T), jnp.int32),
            pltpu.VMEM((2, 3, _BR, _DIM), jnp.float32),
            pltpu.SemaphoreType.DMA,
            pltpu.SemaphoreType.DMA((2,)),
            pltpu.SemaphoreType.DMA((2,)),
        ],
    )(mat, trow, gc2d, z_rows, gamma_rows, z_colsT, idx, b, sigma)


def kernel(batch_events_mat, col_idx_list, batch_events_time,
           gamma_rows, gamma_cols, z_rows, z_cols, b, sigma):
    idx = jnp.pad(col_idx_list.astype(jnp.int32), (0, _TP - _T))
    gpad2 = jnp.pad(gamma_cols, (0, _GPAD - gamma_cols.shape[0])
                    ).reshape(_GPAD // 128, 128)
    gc = _sc_gather(gpad2, idx)
    gc2d = gc[:_T].reshape(1, _T)
    trow = jnp.pad(batch_events_time, (0, _TP - _T)).reshape(1, _TP)
    npad = pl.cdiv(_ROWS, _BR) * _BR - _ROWS
    gr_row = jnp.pad(gamma_rows, (0, npad)).reshape(1, -1)
    total = _tc_call(batch_events_mat.astype(jnp.int32), trow, gc2d,
                     z_rows, gr_row, z_cols.T, idx,
                     b.astype(jnp.float32), sigma)
    return -total[0, 0]


# speedup vs baseline: 2.8823x; 1.5327x over previous
import jax
import jax.numpy as jnp
from jax import lax
from jax.experimental import pallas as pl
from jax.experimental.pallas import tpu as pltpu
from jax.experimental.pallas import tpu_sc as plsc

_BIG = 100000.0
_T = 100
_TP = 128
_DIM = 32
_ROWS = 10000
_BC = 2048
_TAIL = _ROWS - (_ROWS // _BC) * _BC
_NW = 8
_BPW = _TP // _NW
_GPAD = 100096
_COLS = 100000
_INV_SQRT2 = 0.7071067811865476


def _sc_gather_body(gpad_hbm, idx_hbm, gc_out, idxv, igv, grows_v, gcol_v,
                    sem):
    wid = lax.axis_index("s") * 2 + lax.axis_index("c")

    @pl.when(wid < _NW)
    def _():
        base = wid * _BPW
        pltpu.sync_copy(idx_hbm.at[pl.ds(base, _BPW)], idxv)
        iv = idxv[...]
        igv[...] = lax.shift_right_logical(iv, 7)
        pltpu.async_copy(gpad_hbm.at[igv], grows_v, sem).wait()
        rowi = lax.iota(jnp.int32, 16)
        gcol_v[...] = plsc.load_gather(grows_v, [rowi, iv & 127])
        pltpu.sync_copy(gcol_v, gc_out.at[pl.ds(base, _BPW)])


def _sc_gather(gpad2, idx):
    return pl.kernel(
        _sc_gather_body,
        out_type=jax.ShapeDtypeStruct((_TP,), jnp.float32),
        mesh=plsc.VectorSubcoreMesh(core_axis_name="c", subcore_axis_name="s"),
        compiler_params=pltpu.CompilerParams(needs_layout_passes=False),
        scratch_types=[
            pltpu.VMEM((_BPW,), jnp.int32),
            pltpu.VMEM((_BPW,), jnp.int32),
            pltpu.VMEM((_BPW, 128), jnp.float32),
            pltpu.VMEM((_BPW,), jnp.float32),
            pltpu.SemaphoreType.DMA,
        ],
    )(gpad2, idx)


def _tc_body(matT_ref, trow_ref, gc_ref, zrT_ref, gr_ref, zcolsT_hbm,
             idx_ref, b_ref, sigma_ref, out_ref, zcT_vmem, gcoarse_vmem,
             zsem):
    nlast = _COLS - 128
    copies = []
    for j in range(_TP):
        cb = jnp.minimum((idx_ref[j] >> 7) * 128, nlast)
        copies.append(pltpu.make_async_copy(
            zcolsT_hbm.at[:, pl.ds(pl.multiple_of(cb, 128), 128)],
            gcoarse_vmem.at[j], zsem))
    for c in copies:
        c.start()
    for c in copies:
        c.wait()
    lane = lax.broadcasted_iota(jnp.int32, (_DIM, 128), 1)
    for j in range(_TP):
        cb = jnp.minimum((idx_ref[j] >> 7) * 128, nlast)
        m = idx_ref[j] - cb
        col = jnp.sum(jnp.where(lane == m, gcoarse_vmem[j], 0.0),
                      axis=1, keepdims=True)
        zcT_vmem[:, pl.ds(j, 1)] = col

    tr1 = trow_ref[...]
    tr2 = 0.5 * tr1 * tr1
    wvT = zcT_vmem[...] - 1e-6
    ww = jnp.sum(wvT * wvT, axis=0, keepdims=True)
    ones_blk = jnp.ones((_DIM, _TP), jnp.float32)
    W = jnp.concatenate([
        -2.0 * wvT, (-2.0 * tr1) * wvT, (-2.0 * tr2) * wvT,
        ones_blk, (2.0 * tr1) * ones_blk, (2.0 * tr2) * ones_blk,
        (tr1 * tr1) * ones_blk, (2.0 * tr1 * tr2) * ones_blk,
        (tr2 * tr2) * ones_blk,
        ww,
    ], axis=0)

    gc_col = gc_ref[...][:_T]
    th = [-_BIG, b_ref[0], b_ref[1], b_ref[2], b_ref[3], _BIG]
    inv_sigma = 1.0 / sigma_ref[0]

    def chunk_ll(off, w):
        z0 = zrT_ref[0, :, pl.ds(off, w)]
        z1 = zrT_ref[1, :, pl.ds(off, w)]
        z2 = zrT_ref[2, :, pl.ds(off, w)]
        FT = jnp.concatenate([
            z0, z1, z2, z0 * z0, z0 * z1, z0 * z2, z1 * z1, z1 * z2,
            z2 * z2, jnp.ones((1, w), jnp.float32),
        ], axis=0)
        dist2 = lax.dot_general(W, FT, (((0,), (0,)), ((), ())),
                                preferred_element_type=jnp.float32,
                                precision=None)
        dist = jnp.sqrt(jnp.maximum(dist2[:_T], 0.0))
        f = -dist + gc_col + gr_ref[:, pl.ds(off, w)]
        mat = matT_ref[:, pl.ds(off, w)]
        active = mat != 0
        y1 = jnp.where(active, mat, 1)
        thi = jnp.where(y1 == 1, th[1],
              jnp.where(y1 == 2, th[2],
              jnp.where(y1 == 3, th[3], th[4])))
        tlo = jnp.where(y1 == 1, th[0],
              jnp.where(y1 == 2, th[1],
              jnp.where(y1 == 3, th[2], th[3])))
        cdf_hi = 0.5 * (1.0 + lax.erf((thi - f) * inv_sigma * _INV_SQRT2))
        cdf_lo = 0.5 * (1.0 + lax.erf((tlo - f) * inv_sigma * _INV_SQRT2))
        ll = jnp.log(cdf_hi - cdf_lo)
        return jnp.sum(jnp.where(active, ll, 0.0))

    def body(k, acc):
        off = pl.multiple_of(k * _BC, 128)
        return acc + chunk_ll(off, _BC)

    acc = lax.fori_loop(0, _ROWS // _BC, body, jnp.float32(0.0))
    acc = acc + chunk_ll(pl.multiple_of((_ROWS // _BC) * _BC, 128), _TAIL)
    out_ref[0, 0] = acc


def _tc_call(matT, trow, gc_col, z_rowsT, gr_row, z_colsT, idx, b, sigma):
    return pl.pallas_call(
        _tc_body,
        grid=(1,),
        in_specs=[
            pl.BlockSpec((_T, _ROWS), lambda i: (0, 0)),
            pl.BlockSpec((1, _TP), lambda i: (0, 0)),
            pl.BlockSpec((_TP, 1), lambda i: (0, 0)),
            pl.BlockSpec((3, _DIM, _ROWS), lambda i: (0, 0, 0)),
            pl.BlockSpec((1, _ROWS), lambda i: (0, 0)),
            pl.BlockSpec(memory_space=pltpu.MemorySpace.HBM),
            pl.BlockSpec(memory_space=pltpu.SMEM),
            pl.BlockSpec(memory_space=pltpu.SMEM),
            pl.BlockSpec(memory_space=pltpu.SMEM),
        ],
        out_specs=pl.BlockSpec((1, 1), lambda i: (0, 0),
                               memory_space=pltpu.SMEM),
        out_shape=jax.ShapeDtypeStruct((1, 1), jnp.float32),
        scratch_shapes=[
            pltpu.VMEM((_DIM, _TP), jnp.float32),
            pltpu.VMEM((_TP, _DIM, 128), jnp.float32),
            pltpu.SemaphoreType.DMA,
        ],
    )(matT, trow, gc_col, z_rowsT, gr_row, z_colsT, idx, b, sigma)


def kernel(batch_events_mat, col_idx_list, batch_events_time,
           gamma_rows, gamma_cols, z_rows, z_cols, b, sigma):
    idx = jnp.pad(col_idx_list.astype(jnp.int32), (0, _TP - _T))
    gpad2 = jnp.pad(gamma_cols, (0, _GPAD - gamma_cols.shape[0])
                    ).reshape(_GPAD // 128, 128)
    gc = _sc_gather(gpad2, idx)
    gc_col = gc.reshape(_TP, 1)
    trow = jnp.pad(batch_events_time, (0, _TP - _T)).reshape(1, _TP)
    gr_row = gamma_rows.reshape(1, -1)
    total = _tc_call(batch_events_mat.T.astype(jnp.int32), trow, gc_col,
                     z_rows.transpose(0, 2, 1), gr_row, z_cols.T, idx,
                     b.astype(jnp.float32), sigma)
    return -total[0, 0]
